# baseline (speedup 1.0000x reference)
import functools

import jax
import jax.numpy as jnp
from jax import lax
from jax.experimental import pallas as pl
from jax.experimental.pallas import tpu as pltpu
from jax.experimental.pallas import tpu_sc as plsc

VOCAB = 100000
EMBED = 64
B = 4096
T = 200
BFLAT = B * T
HALF = BFLAT // 2
OUT_ROWS = BFLAT // 2


@functools.cache
def _build(num_cores: int, num_subcores: int):
    nw = num_cores * num_subcores
    rows_w = OUT_ROWS // nw
    g = 128
    n_groups = rows_w // g

    mesh = plsc.VectorSubcoreMesh(core_axis_name="c", subcore_axis_name="s")

    @functools.partial(
        pl.kernel,
        out_type=jax.ShapeDtypeStruct((OUT_ROWS, 2 * EMBED), jnp.float32),
        mesh=mesh,
        scratch_types=[
            pltpu.VMEM((2, rows_w), jnp.int32),
            pltpu.VMEM((2, g, EMBED), jnp.float32),
            pltpu.VMEM((2, g, EMBED), jnp.float32),
            pltpu.SemaphoreType.DMA,
            pltpu.SemaphoreType.DMA,
        ],
        compiler_params=pltpu.CompilerParams(use_tc_tiling_on_sc=False),
    )
    def gather_kernel(tok_hbm, table_hbm, out_hbm, idx_v, buf0, buf1, sem0, sem1):
        wid = lax.axis_index("s") * num_cores + lax.axis_index("c")
        base = wid * rows_w
        pltpu.sync_copy(tok_hbm.at[pl.ds(base, rows_w)], idx_v.at[0])
        pltpu.sync_copy(tok_hbm.at[pl.ds(HALF + base, rows_w)], idx_v.at[1])

        def fire(gi, buf, sem):
            pltpu.async_copy(
                table_hbm.at[idx_v.at[0, pl.ds(gi * g, g)]], buf.at[0], sem)
            pltpu.async_copy(
                table_hbm.at[idx_v.at[1, pl.ds(gi * g, g)]], buf.at[1], sem)

        def drain(buf, sem):
            pltpu.make_async_copy(
                table_hbm.at[idx_v.at[0, pl.ds(0, g)]], buf.at[0], sem).wait()
            pltpu.make_async_copy(
                table_hbm.at[idx_v.at[1, pl.ds(0, g)]], buf.at[1], sem).wait()

        def store(gi, buf):
            pltpu.sync_copy(
                buf.at[0], out_hbm.at[pl.ds(base + gi * g, g), pl.ds(0, EMBED)])
            pltpu.sync_copy(
                buf.at[1], out_hbm.at[pl.ds(base + gi * g, g), pl.ds(EMBED, EMBED)])

        fire(0, buf0, sem0)

        @pl.loop(0, n_groups, step=2)
        def _(gi):
            fire(gi + 1, buf1, sem1)
            drain(buf0, sem0)
            store(gi, buf0)

            @pl.when(gi + 2 < n_groups)
            def _():
                fire(gi + 2, buf0, sem0)

            drain(buf1, sem1)
            store(gi + 1, buf1)

    return gather_kernel


def kernel(token_ids, table):
    info = plsc.get_sparse_core_info()
    fn = _build(info.num_cores, info.num_subcores)
    tok = token_ids.astype(jnp.int32).reshape(-1, 2)
    tok = jnp.concatenate([tok[:, 0], tok[:, 1]])
    out = fn(tok, table)
    return out.reshape(B, T, EMBED)


# speedup vs baseline: 1.7796x; 1.7796x over previous
import functools

import jax
import jax.numpy as jnp
from jax import lax
from jax.experimental import pallas as pl
from jax.experimental.pallas import tpu as pltpu
from jax.experimental.pallas import tpu_sc as plsc

VOCAB = 100000
EMBED = 64
B = 4096
T = 200
BFLAT = B * T
ROW = 2 * EMBED


@functools.cache
def _build(num_cores: int, num_subcores: int):
    nw = num_cores * num_subcores
    n_per_w = BFLAT // nw
    g = 128
    n_groups = n_per_w // g

    mesh = plsc.VectorSubcoreMesh(core_axis_name="c", subcore_axis_name="s")

    @functools.partial(
        pl.kernel,
        out_type=jax.ShapeDtypeStruct((BFLAT, ROW), jnp.float32),
        mesh=mesh,
        scratch_types=[
            pltpu.VMEM((n_per_w,), jnp.int32),
            pltpu.VMEM((g, ROW), jnp.float32),
            pltpu.VMEM((g, ROW), jnp.float32),
            pltpu.SemaphoreType.DMA,
            pltpu.SemaphoreType.DMA,
        ],
    )
    def gather_kernel(tok_hbm, table_hbm, out_hbm, idx_v, buf0, buf1, sem0, sem1):
        wid = lax.axis_index("s") * num_cores + lax.axis_index("c")
        base = wid * n_per_w
        pltpu.sync_copy(tok_hbm.at[pl.ds(base, n_per_w)], idx_v)

        def fire(gi, buf, sem):
            pltpu.async_copy(
                table_hbm.at[idx_v.at[pl.ds(gi * g, g)]], buf, sem)

        def drain(buf, sem):
            pltpu.make_async_copy(
                table_hbm.at[idx_v.at[pl.ds(0, g)]], buf, sem).wait()

        def store(gi, buf):
            pltpu.sync_copy(buf, out_hbm.at[pl.ds(base + gi * g, g)])

        fire(0, buf0, sem0)

        @pl.loop(0, n_groups, step=2)
        def _(gi):
            fire(gi + 1, buf1, sem1)
            drain(buf0, sem0)
            store(gi, buf0)

            @pl.when(gi + 2 < n_groups)
            def _():
                fire(gi + 2, buf0, sem0)

            drain(buf1, sem1)
            store(gi + 1, buf1)

    return gather_kernel


def kernel(token_ids, table):
    info = plsc.get_sparse_core_info()
    fn = _build(info.num_cores, info.num_subcores)
    tok = token_ids.astype(jnp.int32).reshape(-1)
    table_padded = jnp.pad(table, ((0, 0), (0, ROW - EMBED)))
    out = fn(tok, table_padded)
    return out[:, :EMBED].reshape(B, T, EMBED)


# speedup vs baseline: 1.7851x; 1.0031x over previous
import functools

import jax
import jax.numpy as jnp
from jax import lax
from jax.experimental import pallas as pl
from jax.experimental.pallas import tpu as pltpu
from jax.experimental.pallas import tpu_sc as plsc

VOCAB = 100000
EMBED = 64
B = 4096
T = 200
BFLAT = B * T
ROW = 2 * EMBED


@functools.cache
def _build(num_cores: int, num_subcores: int):
    nw = num_cores * num_subcores
    n_per_w = BFLAT // nw
    g = 128
    n_groups = n_per_w // g

    mesh = plsc.VectorSubcoreMesh(core_axis_name="c", subcore_axis_name="s")

    @functools.partial(
        pl.kernel,
        out_type=jax.ShapeDtypeStruct((BFLAT, ROW), jnp.float32),
        mesh=mesh,
        scratch_types=[
            pltpu.VMEM((n_per_w,), jnp.int32),
            *([pltpu.VMEM((g, ROW), jnp.float32)] * 4),
            *([pltpu.SemaphoreType.DMA] * 4),
        ],
    )
    def gather_kernel(tok_hbm, table_hbm, out_hbm, idx_v, b0, b1, b2, b3,
                      s0, s1, s2, s3):
        bufs = (b0, b1, b2, b3)
        sems = (s0, s1, s2, s3)
        wid = lax.axis_index("s") * num_cores + lax.axis_index("c")
        base = wid * n_per_w
        pltpu.sync_copy(tok_hbm.at[pl.ds(base, n_per_w)], idx_v)

        def fire(gi, buf, sem):
            pltpu.async_copy(
                table_hbm.at[idx_v.at[pl.ds(gi * g, g)]], buf, sem)

        def drain(buf, sem):
            pltpu.make_async_copy(
                table_hbm.at[idx_v.at[pl.ds(0, g)]], buf, sem).wait()

        def store(gi, buf):
            pltpu.sync_copy(buf, out_hbm.at[pl.ds(base + gi * g, g)])

        for j in range(4):
            fire(j, bufs[j], sems[j])

        @pl.loop(0, n_groups, step=4)
        def _(gi):
            for j in range(4):
                drain(bufs[j], sems[j])
                store(gi + j, bufs[j])

                @pl.when(gi + j + 4 < n_groups)
                def _():
                    fire(gi + j + 4, bufs[j], sems[j])

    return gather_kernel


def kernel(token_ids, table):
    info = plsc.get_sparse_core_info()
    fn = _build(info.num_cores, info.num_subcores)
    tok = token_ids.astype(jnp.int32).reshape(-1)
    table_padded = jnp.pad(table, ((0, 0), (0, ROW - EMBED)))
    out = fn(tok, table_padded)
    return out[:, :EMBED].reshape(B, T, EMBED)
